# baseline (speedup 1.0000x reference)
import functools

import jax
import jax.numpy as jnp
from jax import lax
from jax.experimental import pallas as pl
from jax.experimental.pallas import tpu as pltpu
from jax.experimental.pallas import tpu_sc as plsc

V = 1000000
D = 64
B = 32
N = 32
K = 100
P = B * N
W = 112
NORM_TERM = 13.815510557964274
MIN_PROB = 1e-9

NC = 2
NS = 16
NW = NC * NS
PB = P // NW
KB = W // 16

TPOS = 16


def _sc_gather_body(idx_hbm, emb_hbm, bias_hbm, noise_hbm,
                    rows_out, q_out, b_out,
                    idx_v, q_v, b_v,
                    sem_r, sem_q, sem_b):
    wid = lax.axis_index("s") * NC + lax.axis_index("c")
    base = wid * PB
    pltpu.sync_copy(idx_hbm.at[pl.ds(base, PB)], idx_v)

    def qb_issue(p, c):
        pltpu.async_copy(noise_hbm.at[idx_v.at[p]], q_v.at[p], sem_q)
        pltpu.async_copy(bias_hbm.at[idx_v.at[p]], b_v.at[p], sem_b)
        return c

    lax.fori_loop(0, PB, qb_issue, 0)

    def row_issue(p, c):
        for kb in range(KB):
            ivec = idx_v[p, pl.ds(kb * 16, 16)]
            for j in range(16):
                pltpu.async_copy(
                    emb_hbm.at[ivec[j]],
                    rows_out.at[base + p, kb * 16 + j],
                    sem_r)
        return c

    lax.fori_loop(0, PB, row_issue, 0)

    def qb_drain(p, c):
        pltpu.make_async_copy(noise_hbm.at[pl.ds(0, W)], q_v.at[p],
                              sem_q).wait()
        pltpu.make_async_copy(bias_hbm.at[pl.ds(0, W)], b_v.at[p],
                              sem_b).wait()
        return c

    lax.fori_loop(0, PB, qb_drain, 0)
    pltpu.sync_copy(q_v, q_out.at[pl.ds(base, PB)])
    pltpu.sync_copy(b_v, b_out.at[pl.ds(base, PB)])

    def row_drain(i, c):
        pltpu.make_async_copy(
            emb_hbm.at[0], rows_out.at[0, 0], sem_r).wait()
        return c

    lax.fori_loop(0, PB * W, row_drain, 0)


def _tc_dot_body(rows_ref, h_ref, o_ref):
    r = rows_ref[...]
    h = h_ref[...]
    o_ref[...] = jnp.sum(r * h[:, None, :], axis=2)


def _tc_epilogue_body(s_ref, q_ref, b_ref, o_ref):
    s = s_ref[...] + b_ref[...]
    q = q_ref[...]
    p = jnp.clip(jnp.exp(s - NORM_TERM), MIN_PROB, 1.0)
    pt = p / (p + float(K) * q)
    col = lax.broadcasted_iota(jnp.int32, s.shape, 1)
    logp = jnp.maximum(jnp.log(pt), -100.0)
    log1mp = jnp.maximum(jnp.log(1.0 - pt), -100.0)
    bce = jnp.where(col == 0, -logp, -log1mp)
    bce = jnp.where(col < K + 1, bce, 0.0)
    o_ref[...] = (jnp.sum(bce) * (1.0 / P)).reshape(1, 1)


def kernel(target, inp, noise_samples, noise, emb_weight, emb_bias):
    idx = jnp.concatenate(
        [target.reshape(P, 1), noise_samples.reshape(P, K)], axis=1)
    idx = jnp.concatenate(
        [idx, jnp.zeros((P, W - (K + 1)), jnp.int32)], axis=1).astype(jnp.int32)
    inp2d = inp.reshape(P, D).astype(jnp.float32)

    mesh = plsc.VectorSubcoreMesh(core_axis_name="c", subcore_axis_name="s")
    sc = pl.kernel(
        _sc_gather_body,
        mesh=mesh,
        compiler_params=pltpu.CompilerParams(
            needs_layout_passes=False, use_tc_tiling_on_sc=True),
        out_type=[
            jax.ShapeDtypeStruct((P, W, D), jnp.float32),
            jax.ShapeDtypeStruct((P, W), jnp.float32),
            jax.ShapeDtypeStruct((P, W), jnp.float32),
        ],
        scratch_types=[
            pltpu.VMEM((PB, W), jnp.int32),
            pltpu.VMEM((PB, W), jnp.float32),
            pltpu.VMEM((PB, W), jnp.float32),
            pltpu.SemaphoreType.DMA,
            pltpu.SemaphoreType.DMA,
            pltpu.SemaphoreType.DMA,
        ],
    )
    rows, qvals, bvals = sc(idx, emb_weight, emb_bias, noise)

    scores = pl.pallas_call(
        _tc_dot_body,
        grid=(P // TPOS,),
        in_specs=[
            pl.BlockSpec((TPOS, W, D), lambda i: (i, 0, 0)),
            pl.BlockSpec((TPOS, D), lambda i: (i, 0)),
        ],
        out_specs=pl.BlockSpec((TPOS, W), lambda i: (i, 0)),
        out_shape=jax.ShapeDtypeStruct((P, W), jnp.float32),
    )(rows, inp2d)

    out = pl.pallas_call(
        _tc_epilogue_body,
        out_shape=jax.ShapeDtypeStruct((1, 1), jnp.float32),
    )(scores, qvals, bvals)
    return out[0, 0]


# speedup vs baseline: 2.1479x; 2.1479x over previous
import functools

import jax
import jax.numpy as jnp
from jax import lax
from jax.experimental import pallas as pl
from jax.experimental.pallas import tpu as pltpu
from jax.experimental.pallas import tpu_sc as plsc

V = 1000000
D = 64
B = 32
N = 32
K = 100
P = B * N
W = 112
NORM_TERM = 13.815510557964274
MIN_PROB = 1e-9

NC = 2
NS = 16
NW = NC * NS
PB = P // NW
KB = W // 16
NB = PB * KB
NRS = 4
G = 8


def _sc_kernel_body(idx_hbm, inp_hbm, emb_hbm, bias_hbm, noise_hbm,
                    s_out, q_out, b_out,
                    idx_v, h_v, rows_v, q_v, b_v, s_v, sp_v,
                    sem_r, sem_q, sem_b):
    wid = lax.axis_index("s") * NC + lax.axis_index("c")
    base = wid * PB
    pltpu.sync_copy(idx_hbm.at[pl.ds(base, PB)], idx_v)
    pltpu.sync_copy(inp_hbm.at[pl.ds(base, PB)], h_v)

    lane = lax.iota(jnp.int32, 16)

    for p in range(PB):
        pltpu.async_copy(noise_hbm.at[idx_v.at[p]], q_v.at[p], sem_q)
        pltpu.async_copy(bias_hbm.at[idx_v.at[p]], b_v.at[p], sem_b)

    def issue_block(pt, kbt, slot):
        ivec = idx_v[pt, pl.ds(kbt * 16, 16)]
        for j in range(16):
            gb = pl.multiple_of(ivec[j] & (-G), G)
            pltpu.async_copy(
                emb_hbm.at[pl.ds(gb, G)],
                rows_v.at[slot, j],
                sem_r.at[slot])

    for t in range(NRS - 1):
        issue_block(t // KB, t % KB, t)

    def blk_body(i, carry):
        p, kb, pt, kbt = carry
        slot = i & (NRS - 1)

        @pl.when(i + NRS - 1 < NB)
        def _():
            issue_block(pt, kbt, (i + NRS - 1) & (NRS - 1))

        for _ in range(16):
            pltpu.make_async_copy(
                emb_hbm.at[pl.ds(0, G)],
                rows_v.at[slot, 0],
                sem_r.at[slot]).wait()

        hs = [h_v[p, pl.ds(t16 * 16, 16)] for t16 in range(D // 16)]
        ivec = idx_v[p, pl.ds(kb * 16, 16)]
        svec = jnp.zeros((16,), jnp.float32)
        for j in range(16):
            o = ivec[j] & (G - 1)
            acc = rows_v[slot, j, o, pl.ds(0, 16)] * hs[0]
            for t in range(1, D // 16):
                acc = acc + rows_v[slot, j, o, pl.ds(t * 16, 16)] * hs[t]
            svec = jnp.where(lane == j, jnp.sum(acc), svec)
        s_v[i] = svec

        kb2 = kb + 1
        wrap = kb2 == KB
        p2 = jnp.where(wrap, p + 1, p)
        kb2 = jnp.where(wrap, 0, kb2)
        kbt2 = kbt + 1
        wrapt = kbt2 == KB
        pt2 = jnp.where(wrapt, pt + 1, pt)
        kbt2 = jnp.where(wrapt, 0, kbt2)
        return (p2, kb2, pt2, kbt2)

    lax.fori_loop(0, NB, blk_body,
                  (0, 0, (NRS - 1) // KB, (NRS - 1) % KB))

    for p in range(PB):
        for kb in range(KB):
            sp_v[p, pl.ds(kb * 16, 16)] = s_v[p * KB + kb]

    for p in range(PB):
        pltpu.make_async_copy(noise_hbm.at[pl.ds(0, W)], q_v.at[p],
                              sem_q).wait()
        pltpu.make_async_copy(bias_hbm.at[pl.ds(0, W)], b_v.at[p],
                              sem_b).wait()
    pltpu.sync_copy(sp_v, s_out.at[pl.ds(base, PB)])
    pltpu.sync_copy(q_v, q_out.at[pl.ds(base, PB)])
    pltpu.sync_copy(b_v, b_out.at[pl.ds(base, PB)])


def _tc_epilogue_body(s_ref, q_ref, b_ref, o_ref):
    s = s_ref[...] + b_ref[...]
    q = q_ref[...]
    p = jnp.clip(jnp.exp(s - NORM_TERM), MIN_PROB, 1.0)
    pt = p / (p + float(K) * q)
    col = lax.broadcasted_iota(jnp.int32, s.shape, 1)
    logp = jnp.maximum(jnp.log(pt), -100.0)
    log1mp = jnp.maximum(jnp.log(1.0 - pt), -100.0)
    bce = jnp.where(col == 0, -logp, -log1mp)
    bce = jnp.where(col < K + 1, bce, 0.0)
    o_ref[...] = (jnp.sum(bce) * (1.0 / P)).reshape(1, 1)


def kernel(target, inp, noise_samples, noise, emb_weight, emb_bias):
    idx = jnp.concatenate(
        [target.reshape(P, 1), noise_samples.reshape(P, K)], axis=1)
    idx = jnp.concatenate(
        [idx, jnp.zeros((P, W - (K + 1)), jnp.int32)], axis=1).astype(jnp.int32)
    inp2d = inp.reshape(P, D).astype(jnp.float32)

    mesh = plsc.VectorSubcoreMesh(core_axis_name="c", subcore_axis_name="s")
    sc = pl.kernel(
        _sc_kernel_body,
        mesh=mesh,
        compiler_params=pltpu.CompilerParams(
            needs_layout_passes=False, use_tc_tiling_on_sc=True),
        out_type=[
            jax.ShapeDtypeStruct((P, W), jnp.float32),
            jax.ShapeDtypeStruct((P, W), jnp.float32),
            jax.ShapeDtypeStruct((P, W), jnp.float32),
        ],
        scratch_types=[
            pltpu.VMEM((PB, W), jnp.int32),
            pltpu.VMEM((PB, D), jnp.float32),
            pltpu.VMEM((NRS, 16, G, D), jnp.float32),
            pltpu.VMEM((PB, W), jnp.float32),
            pltpu.VMEM((PB, W), jnp.float32),
            pltpu.VMEM((NB, 16), jnp.float32),
            pltpu.VMEM((PB, W), jnp.float32),
            pltpu.SemaphoreType.DMA((NRS,)),
            pltpu.SemaphoreType.DMA,
            pltpu.SemaphoreType.DMA,
        ],
    )
    scores, qvals, bvals = sc(idx, inp2d, emb_weight, emb_bias, noise)

    out = pl.pallas_call(
        _tc_epilogue_body,
        out_shape=jax.ShapeDtypeStruct((1, 1), jnp.float32),
    )(scores, qvals, bvals)
    return out[0, 0]


# speedup vs baseline: 2.5751x; 1.1989x over previous
import functools

import jax
import jax.numpy as jnp
from jax import lax
from jax.experimental import pallas as pl
from jax.experimental.pallas import tpu as pltpu
from jax.experimental.pallas import tpu_sc as plsc

V = 1000000
D = 64
B = 32
N = 32
K = 100
P = B * N
W = 112
NORM_TERM = 13.815510557964274
MIN_PROB = 1e-9

NC = 2
NS = 16
NW = NC * NS
PB = P // NW
KB = W // 16
RS = 6


def _sc_kernel_body(idx_hbm, inp_hbm, emb_hbm, bias_hbm, noise_hbm,
                    s_out, q_out, b_out,
                    idx_v, h_v, rows_v, q_v, b_v, s_v,
                    sem_r, sem_q, sem_b):
    wid = lax.axis_index("s") * NC + lax.axis_index("c")
    base = wid * PB
    pltpu.sync_copy(idx_hbm.at[pl.ds(base, PB)], idx_v)
    pltpu.sync_copy(inp_hbm.at[pl.ds(base, PB)], h_v)

    lane = lax.iota(jnp.int32, 16)

    for p in range(PB):
        pltpu.async_copy(noise_hbm.at[idx_v.at[p]], q_v.at[p], sem_q)
        pltpu.async_copy(bias_hbm.at[idx_v.at[p]], b_v.at[p], sem_b)

    def issue_rows(p, slot):
        for kb in range(KB):
            ivec = idx_v[p, pl.ds(kb * 16, 16)]
            for j in range(16):
                pltpu.async_copy(
                    emb_hbm.at[ivec[j]],
                    rows_v.at[slot, kb * 16 + j],
                    sem_r.at[slot])

    def wait_rows(slot):
        for _ in range(W):
            pltpu.make_async_copy(
                emb_hbm.at[0],
                rows_v.at[slot, 0],
                sem_r.at[slot]).wait()

    for p in range(RS - 1):
        issue_rows(p, p % RS)

    def pos_body(p, carry):
        slot = lax.rem(p, RS)

        @pl.when(p + RS - 1 < PB)
        def _():
            issue_rows(p + RS - 1, lax.rem(p + RS - 1, RS))

        wait_rows(slot)

        hs = [h_v[p, pl.ds(j * 16, 16)] for j in range(D // 16)]
        for kb in range(KB):
            svec = jnp.zeros((16,), jnp.float32)
            for j in range(16):
                k = kb * 16 + j
                acc = rows_v[slot, k, pl.ds(0, 16)] * hs[0]
                for t in range(1, D // 16):
                    acc = acc + rows_v[slot, k, pl.ds(t * 16, 16)] * hs[t]
                svec = jnp.where(lane == j, jnp.sum(acc), svec)
            s_v[p, pl.ds(kb * 16, 16)] = svec
        return carry

    lax.fori_loop(0, PB, pos_body, 0)

    for p in range(PB):
        pltpu.make_async_copy(noise_hbm.at[pl.ds(0, W)], q_v.at[p],
                              sem_q).wait()
        pltpu.make_async_copy(bias_hbm.at[pl.ds(0, W)], b_v.at[p],
                              sem_b).wait()
    pltpu.sync_copy(s_v, s_out.at[pl.ds(base, PB)])
    pltpu.sync_copy(q_v, q_out.at[pl.ds(base, PB)])
    pltpu.sync_copy(b_v, b_out.at[pl.ds(base, PB)])


def _tc_epilogue_body(s_ref, q_ref, b_ref, o_ref):
    s = s_ref[...] + b_ref[...]
    q = q_ref[...]
    p = jnp.clip(jnp.exp(s - NORM_TERM), MIN_PROB, 1.0)
    pt = p / (p + float(K) * q)
    col = lax.broadcasted_iota(jnp.int32, s.shape, 1)
    logp = jnp.maximum(jnp.log(pt), -100.0)
    log1mp = jnp.maximum(jnp.log(1.0 - pt), -100.0)
    bce = jnp.where(col == 0, -logp, -log1mp)
    bce = jnp.where(col < K + 1, bce, 0.0)
    o_ref[...] = (jnp.sum(bce) * (1.0 / P)).reshape(1, 1)


def kernel(target, inp, noise_samples, noise, emb_weight, emb_bias):
    idx = jnp.concatenate(
        [target.reshape(P, 1), noise_samples.reshape(P, K)], axis=1)
    idx = jnp.concatenate(
        [idx, jnp.zeros((P, W - (K + 1)), jnp.int32)], axis=1).astype(jnp.int32)
    inp2d = inp.reshape(P, D).astype(jnp.float32)

    mesh = plsc.VectorSubcoreMesh(core_axis_name="c", subcore_axis_name="s")
    sc = pl.kernel(
        _sc_kernel_body,
        mesh=mesh,
        compiler_params=pltpu.CompilerParams(
            needs_layout_passes=False, use_tc_tiling_on_sc=True),
        out_type=[
            jax.ShapeDtypeStruct((P, W), jnp.float32),
            jax.ShapeDtypeStruct((P, W), jnp.float32),
            jax.ShapeDtypeStruct((P, W), jnp.float32),
        ],
        scratch_types=[
            pltpu.VMEM((PB, W), jnp.int32),
            pltpu.VMEM((PB, D), jnp.float32),
            pltpu.VMEM((RS, W, D), jnp.float32),
            pltpu.VMEM((PB, W), jnp.float32),
            pltpu.VMEM((PB, W), jnp.float32),
            pltpu.VMEM((PB, W), jnp.float32),
            pltpu.SemaphoreType.DMA((RS,)),
            pltpu.SemaphoreType.DMA,
            pltpu.SemaphoreType.DMA,
        ],
    )
    scores, qvals, bvals = sc(idx, inp2d, emb_weight, emb_bias, noise)

    out = pl.pallas_call(
        _tc_epilogue_body,
        out_shape=jax.ShapeDtypeStruct((1, 1), jnp.float32),
    )(scores, qvals, bvals)
    return out[0, 0]


# speedup vs baseline: 2.5776x; 1.0010x over previous
import functools

import jax
import jax.numpy as jnp
from jax import lax
from jax.experimental import pallas as pl
from jax.experimental.pallas import tpu as pltpu
from jax.experimental.pallas import tpu_sc as plsc

V = 1000000
D = 64
B = 32
N = 32
K = 100
P = B * N
W = 112
NORM_TERM = 13.815510557964274
MIN_PROB = 1e-9

NC = 2
NS = 16
NW = NC * NS
PB = P // NW
KB = W // 16
RS = 4


def _sc_kernel_body(idx_hbm, inp_hbm, emb_hbm, bias_hbm, noise_hbm,
                    s_out, q_out, b_out,
                    idx_v, h_v, rows_v, q_v, b_v, s_v,
                    sem_r, sem_q, sem_b):
    wid = lax.axis_index("s") * NC + lax.axis_index("c")
    base = wid * PB
    pltpu.sync_copy(idx_hbm.at[pl.ds(base, PB)], idx_v)
    pltpu.sync_copy(inp_hbm.at[pl.ds(base, PB)], h_v)

    lane = lax.iota(jnp.int32, 16)

    for p in range(PB):
        pltpu.async_copy(noise_hbm.at[idx_v.at[p]], q_v.at[p], sem_q)
        pltpu.async_copy(bias_hbm.at[idx_v.at[p]], b_v.at[p], sem_b)

    def issue_rows(p, slot):
        for kb in range(KB):
            ivec = idx_v[p, pl.ds(kb * 16, 16)]
            for j in range(16):
                pltpu.async_copy(
                    emb_hbm.at[ivec[j]],
                    rows_v.at[slot, kb * 16 + j],
                    sem_r.at[slot])

    def wait_rows(slot):
        for _ in range(W):
            pltpu.make_async_copy(
                emb_hbm.at[0],
                rows_v.at[slot, 0],
                sem_r.at[slot]).wait()

    for p in range(RS - 1):
        issue_rows(p, p % RS)

    def pos_body(p, carry):
        slot = lax.rem(p, RS)

        @pl.when(p + RS - 1 < PB)
        def _():
            issue_rows(p + RS - 1, lax.rem(p + RS - 1, RS))

        wait_rows(slot)

        hs = [h_v[p, pl.ds(j * 16, 16)] for j in range(D // 16)]
        for kb in range(KB):
            svec = jnp.zeros((16,), jnp.float32)
            for j in range(16):
                k = kb * 16 + j
                acc = rows_v[slot, k, pl.ds(0, 16)] * hs[0]
                for t in range(1, D // 16):
                    acc = acc + rows_v[slot, k, pl.ds(t * 16, 16)] * hs[t]
                svec = jnp.where(lane == j, jnp.sum(acc), svec)
            s_v[p, pl.ds(kb * 16, 16)] = svec
        return carry

    lax.fori_loop(0, PB, pos_body, 0)

    for p in range(PB):
        pltpu.make_async_copy(noise_hbm.at[pl.ds(0, W)], q_v.at[p],
                              sem_q).wait()
        pltpu.make_async_copy(bias_hbm.at[pl.ds(0, W)], b_v.at[p],
                              sem_b).wait()
    pltpu.sync_copy(s_v, s_out.at[pl.ds(base, PB)])
    pltpu.sync_copy(q_v, q_out.at[pl.ds(base, PB)])
    pltpu.sync_copy(b_v, b_out.at[pl.ds(base, PB)])


def _tc_epilogue_body(s_ref, q_ref, b_ref, o_ref):
    s = s_ref[...] + b_ref[...]
    q = q_ref[...]
    p = jnp.clip(jnp.exp(s - NORM_TERM), MIN_PROB, 1.0)
    pt = p / (p + float(K) * q)
    col = lax.broadcasted_iota(jnp.int32, s.shape, 1)
    logp = jnp.maximum(jnp.log(pt), -100.0)
    log1mp = jnp.maximum(jnp.log(1.0 - pt), -100.0)
    bce = jnp.where(col == 0, -logp, -log1mp)
    bce = jnp.where(col < K + 1, bce, 0.0)
    o_ref[...] = (jnp.sum(bce) * (1.0 / P)).reshape(1, 1)


def kernel(target, inp, noise_samples, noise, emb_weight, emb_bias):
    idx = jnp.concatenate(
        [target.reshape(P, 1), noise_samples.reshape(P, K)], axis=1)
    idx = jnp.concatenate(
        [idx, jnp.zeros((P, W - (K + 1)), jnp.int32)], axis=1).astype(jnp.int32)
    inp2d = inp.reshape(P, D).astype(jnp.float32)

    mesh = plsc.VectorSubcoreMesh(core_axis_name="c", subcore_axis_name="s")
    sc = pl.kernel(
        _sc_kernel_body,
        mesh=mesh,
        compiler_params=pltpu.CompilerParams(
            needs_layout_passes=False, use_tc_tiling_on_sc=True),
        out_type=[
            jax.ShapeDtypeStruct((P, W), jnp.float32),
            jax.ShapeDtypeStruct((P, W), jnp.float32),
            jax.ShapeDtypeStruct((P, W), jnp.float32),
        ],
        scratch_types=[
            pltpu.VMEM((PB, W), jnp.int32),
            pltpu.VMEM((PB, D), jnp.float32),
            pltpu.VMEM((RS, W, D), jnp.float32),
            pltpu.VMEM((PB, W), jnp.float32),
            pltpu.VMEM((PB, W), jnp.float32),
            pltpu.VMEM((PB, W), jnp.float32),
            pltpu.SemaphoreType.DMA((RS,)),
            pltpu.SemaphoreType.DMA,
            pltpu.SemaphoreType.DMA,
        ],
    )
    scores, qvals, bvals = sc(idx, inp2d, emb_weight, emb_bias, noise)

    out = pl.pallas_call(
        _tc_epilogue_body,
        out_shape=jax.ShapeDtypeStruct((1, 1), jnp.float32),
    )(scores, qvals, bvals)
    return out[0, 0]
